# baseline (speedup 1.0000x reference)
import functools

import jax
import jax.numpy as jnp
from jax import lax
from jax.experimental import pallas as pl
from jax.experimental.pallas import tpu as pltpu
from jax.experimental.pallas import tpu_sc as plsc

B = 4096
L = 50
EMB = 64
LABEL = 30
NEG = 5

NC = 2
NS = 16
NW = NC * NS
BPW = B // NW
NLANE = 16
KSUB = EMB // NLANE


def _accum(acc, buf):
  def body(r, carry):
    for k in range(KSUB):
      sl = pl.ds(k * NLANE, NLANE)
      plsc.addupdate(acc.at[r, sl], buf[r, sl])
    return carry

  lax.fori_loop(0, BPW, body, 0)


def _pool_body(xt_hbm, table_hbm, out_hbm, idx_all, buf0, buf1, acc,
               sem0, sem1):
  wid = lax.axis_index("s") * NC + lax.axis_index("c")
  base = wid * BPW

  pltpu.sync_copy(xt_hbm.at[:, pl.ds(base, BPW)], idx_all)

  def zero(r, carry):
    for k in range(KSUB):
      acc[r, pl.ds(k * NLANE, NLANE)] = jnp.zeros((NLANE,), jnp.float32)
    return carry

  lax.fori_loop(0, BPW, zero, 0)

  def gather(l, buf, sem):
    return pltpu.make_async_copy(table_hbm.at[idx_all.at[l]], buf, sem)

  gather(0, buf0, sem0).start()

  def step(i, carry):
    l0 = 2 * i
    gather(l0 + 1, buf1, sem1).start()
    gather(l0, buf0, sem0).wait()
    _accum(acc, buf0)

    @pl.when(l0 + 2 < L)
    def _():
      gather(l0 + 2, buf0, sem0).start()

    gather(l0 + 1, buf1, sem1).wait()
    _accum(acc, buf1)
    return carry

  lax.fori_loop(0, L // 2, step, 0)

  pltpu.sync_copy(acc, out_hbm.at[pl.ds(base, BPW)])


@jax.jit
def _pool(xt, table):
  mesh = plsc.VectorSubcoreMesh(
      core_axis_name="c", subcore_axis_name="s",
      num_cores=NC, num_subcores=NS)
  f = pl.kernel(
      _pool_body,
      out_type=jax.ShapeDtypeStruct((B, EMB), jnp.float32),
      mesh=mesh,
      compiler_params=pltpu.CompilerParams(use_tc_tiling_on_sc=False),
      scratch_types=[
          pltpu.VMEM((L, BPW), jnp.int32),
          pltpu.VMEM((BPW, EMB), jnp.float32),
          pltpu.VMEM((BPW, EMB), jnp.float32),
          pltpu.VMEM((BPW, EMB), jnp.float32),
          pltpu.SemaphoreType.DMA,
          pltpu.SemaphoreType.DMA,
      ],
  )
  return f(xt, table)


def _dense_body(pooled_ref, mask_ref, y_ref, ob_ref, negt_ref, w_ref,
                logit_ref, loss_ref):
  x_len = jnp.sum(mask_ref[...], axis=1, keepdims=True)
  user = pooled_ref[...] / x_len
  logit = lax.dot_general(user, w_ref[...],
                          (((1,), (1,)), ((), ())),
                          preferred_element_type=jnp.float32)
  logit_ref[...] = logit
  ob = ob_ref[...]
  wc = logit * ob
  yc = y_ref[...] * ob
  negsum = negt_ref[0]
  for n in range(1, NEG):
    negsum = negsum + negt_ref[n]
  neg_term = jnp.log(jax.nn.sigmoid(-(negsum * wc)))
  total_neg = jnp.sum(neg_term)
  pos_in = jnp.sum(wc * yc, axis=1)
  pos_loss = jnp.sum(jnp.log(jax.nn.sigmoid(pos_in)))
  loss = -(LABEL * pos_loss + total_neg) / B
  loss_ref[...] = jnp.full((8, 128), loss, jnp.float32)


@jax.jit
def _dense(pooled, x_mask, y, ob, neg_t, w):
  return pl.pallas_call(
      _dense_body,
      out_shape=[
          jax.ShapeDtypeStruct((B, LABEL), jnp.float32),
          jax.ShapeDtypeStruct((8, 128), jnp.float32),
      ],
  )(pooled, x_mask, y, ob, neg_t, w)


def kernel(x, x_mask, y, ob, neg_samples, emb_table, W):
  xt = jnp.transpose(x)
  pooled = _pool(xt, emb_table)
  neg_t = jnp.transpose(neg_samples, (1, 0, 2))
  logit, loss_tile = _dense(pooled, x_mask, y, ob, neg_t, W)
  return logit, loss_tile[0, 0]
